# baseline (speedup 1.0000x reference)
import functools

import jax
import jax.numpy as jnp
from jax import lax
from jax.experimental import pallas as pl
from jax.experimental.pallas import tpu as pltpu
from jax.experimental.pallas import tpu_sc as plsc

ROWS = 8192
DIM = 4096
NC = 2
NS = 16
L = 16
NW = NC * NS
ROWS_PER_W = ROWS // NW
RB = 8
HD = DIM // 2
NBLK = ROWS_PER_W // RB


@functools.partial(
    pl.kernel,
    out_type=jax.ShapeDtypeStruct((ROWS, DIM), jnp.float32),
    mesh=plsc.VectorSubcoreMesh(core_axis_name="c", subcore_axis_name="s"),
    compiler_params=pltpu.CompilerParams(needs_layout_passes=False),
    scratch_types=[
        pltpu.VMEM((DIM,), jnp.int32),
        pltpu.VMEM((RB, DIM), jnp.float32),
        pltpu.VMEM((RB, DIM), jnp.float32),
        pltpu.VMEM((RB, HD), jnp.float32),
        pltpu.VMEM((RB, HD), jnp.float32),
        pltpu.SemaphoreType.DMA,
        pltpu.SemaphoreType.DMA,
        pltpu.SemaphoreType.DMA,
        pltpu.SemaphoreType.DMA,
    ],
)
def _permute(x_hbm, perm_hbm, out_hbm, perm_v, in0, in1, out0, out1,
             si0, si1, so0, so1):
    wid = lax.axis_index("s") * NC + lax.axis_index("c")
    base = wid * ROWS_PER_W
    ins, outs = [in0, in1], [out0, out1]
    sin, sout = [si0, si1], [so0, so1]

    pltpu.sync_copy(perm_hbm, perm_v)

    def start_in(b, k):
        pltpu.async_copy(x_hbm.at[pl.ds(base + b * RB, RB)], ins[k], sin[k])

    def wait_in(k):
        pltpu.make_async_copy(
            x_hbm.at[pl.ds(base, RB)], ins[k], sin[k]).wait()

    def start_out(b, h):
        pltpu.async_copy(
            outs[h],
            out_hbm.at[pl.ds(base + b * RB, RB), pl.ds(h * HD, HD)],
            sout[h])

    def wait_out(h):
        pltpu.make_async_copy(
            outs[h], out_hbm.at[pl.ds(base, RB), pl.ds(0, HD)],
            sout[h]).wait()

    def gather_half(src, h):
        dst = outs[h]

        @plsc.parallel_loop(0, HD // L, unroll=8)
        def _(j):
            pvec = perm_v[pl.ds(h * HD + j * L, L)]
            for r in range(RB):
                ridx = jnp.full((L,), r, jnp.int32)
                dst[r, pl.ds(j * L, L)] = plsc.load_gather(src, [ridx, pvec])

    start_in(0, 0)
    start_in(1, 1)

    def pair_body(i, carry):
        for k in (0, 1):
            b = i * 2 + k
            wait_in(k)
            for h in (0, 1):
                @pl.when(b >= 1)
                def _():
                    wait_out(h)
                gather_half(ins[k], h)
                start_out(b, h)

            @pl.when(b + 2 < NBLK)
            def _():
                start_in(b + 2, k)
        return carry

    lax.fori_loop(0, NBLK // 2, pair_body, 0)
    wait_out(0)
    wait_out(1)


def kernel(x, perm):
    out = _permute(x, perm.astype(jnp.int32))
    return (out, 0.0)


# speedup vs baseline: 1.0019x; 1.0019x over previous
import functools

import jax
import jax.numpy as jnp
from jax import lax
from jax.experimental import pallas as pl
from jax.experimental.pallas import tpu as pltpu
from jax.experimental.pallas import tpu_sc as plsc

ROWS = 8192
DIM = 4096
NC = 2
NS = 16
L = 16
NW = NC * NS
ROWS_PER_W = ROWS // NW
RB = 8
HD = DIM // 2
NBLK = ROWS_PER_W // RB


@functools.partial(
    pl.kernel,
    out_type=jax.ShapeDtypeStruct((ROWS, DIM), jnp.float32),
    mesh=plsc.VectorSubcoreMesh(core_axis_name="c", subcore_axis_name="s"),
    compiler_params=pltpu.CompilerParams(needs_layout_passes=False),
    scratch_types=[
        pltpu.VMEM((DIM,), jnp.int32),
        pltpu.VMEM((RB, DIM), jnp.float32),
        pltpu.VMEM((RB, DIM), jnp.float32),
        pltpu.VMEM((RB, HD), jnp.float32),
        pltpu.VMEM((RB, HD), jnp.float32),
        pltpu.SemaphoreType.DMA,
        pltpu.SemaphoreType.DMA,
        pltpu.SemaphoreType.DMA,
        pltpu.SemaphoreType.DMA,
    ],
)
def _permute(x_hbm, perm_hbm, out_hbm, perm_v, in0, in1, out0, out1,
             si0, si1, so0, so1):
    wid = lax.axis_index("s") * NC + lax.axis_index("c")
    base = wid * ROWS_PER_W
    ins, outs = [in0, in1], [out0, out1]
    sin, sout = [si0, si1], [so0, so1]

    pltpu.sync_copy(perm_hbm, perm_v)

    def start_in(b, k):
        pltpu.async_copy(x_hbm.at[pl.ds(base + b * RB, RB)], ins[k], sin[k])

    def wait_in(k):
        pltpu.make_async_copy(
            x_hbm.at[pl.ds(base, RB)], ins[k], sin[k]).wait()

    def start_out(b, h):
        pltpu.async_copy(
            outs[h],
            out_hbm.at[pl.ds(base + b * RB, RB), pl.ds(h * HD, HD)],
            sout[h])

    def wait_out(h):
        pltpu.make_async_copy(
            outs[h], out_hbm.at[pl.ds(base, RB), pl.ds(0, HD)],
            sout[h]).wait()

    def gather_half(src, h):
        dst = outs[h]

        @plsc.parallel_loop(0, HD // L, unroll=4)
        def _(j):
            pvec = perm_v[pl.ds(h * HD + j * L, L)]
            for r in range(RB):
                ridx = jnp.full((L,), r, jnp.int32)
                dst[r, pl.ds(j * L, L)] = plsc.load_gather(src, [ridx, pvec])

    start_in(0, 0)
    start_in(1, 1)

    def pair_body(i, carry):
        for k in (0, 1):
            b = i * 2 + k
            wait_in(k)
            for h in (0, 1):
                @pl.when(b >= 1)
                def _():
                    wait_out(h)
                gather_half(ins[k], h)
                start_out(b, h)

            @pl.when(b + 2 < NBLK)
            def _():
                start_in(b + 2, k)
        return carry

    lax.fori_loop(0, NBLK // 2, pair_body, 0)
    wait_out(0)
    wait_out(1)


def kernel(x, perm):
    out = _permute(x, perm.astype(jnp.int32))
    return (out, 0.0)
